# baseline (speedup 1.0000x reference)
import jax
import jax.numpy as jnp
from jax import lax
from jax.experimental import pallas as pl
from jax.experimental.pallas import tpu as pltpu

E = 8
TOP_K = 2
D_IN = 1024
D_H = 4096
D_OUT = 1024

T_FFN = 128
T_ROUTE = 512


def _routing_body(x_ref, mus_ref, sig_ref, lss_ref, lp_ref, w_ref, ti_ref, g_ref,
                  xb_ref):
    x = x_ref[...]
    rows = x.shape[0]
    d = jnp.zeros((rows, E), jnp.float32)
    eidx = jax.lax.broadcasted_iota(jnp.int32, (rows, E), 1)
    for e in range(E):
        t = (x - mus_ref[pl.ds(e, 1), :]) * sig_ref[pl.ds(e, 1), :]
        de = jnp.sum(t * t, axis=1, keepdims=True)
        d = jnp.where(eidx == e, de, d)
    lp = -0.5 * d - lss_ref[...]
    m = jnp.max(lp, axis=1, keepdims=True)
    ew = jnp.exp(lp - m)
    w = ew / jnp.sum(ew, axis=1, keepdims=True)

    iota = jax.lax.broadcasted_iota(jnp.int32, w.shape, 1)
    m1 = jnp.max(w, axis=1, keepdims=True)
    i1 = jnp.min(jnp.where(w == m1, iota, E), axis=1, keepdims=True)
    wm = jnp.where(iota == i1, -jnp.inf, w)
    m2 = jnp.max(wm, axis=1, keepdims=True)
    i2 = jnp.min(jnp.where(wm == m2, iota, E), axis=1, keepdims=True)

    lp_ref[...] = lp
    w_ref[...] = w
    ti_ref[...] = jnp.where(iota == 0, i1, jnp.where(iota == 1, i2, 0))
    g_ref[...] = jnp.where(iota == 0, m1, jnp.where(iota == 1, m2, 0.0))
    xb_ref[...] = x.astype(jnp.bfloat16)


def _dispatch_body(ti_ref, pos_ref, meta_ref):
    C = 512
    n = ti_ref.shape[0]
    e0 = ti_ref[:, 0:1]
    e1 = ti_ref[:, 1:2]
    lane8 = lax.broadcasted_iota(jnp.int32, (n, E), 1)
    oh0 = (e0 == lane8).astype(jnp.float32)
    oh1 = (e1 == lane8).astype(jnp.float32)
    oh = oh0 + oh1

    r = lax.broadcasted_iota(jnp.int32, (C, C), 0)
    c = lax.broadcasted_iota(jnp.int32, (C, C), 1)
    L = (r > c).astype(jnp.float32)

    carry = jnp.zeros((1, E), jnp.float32)
    chunks = []
    for k in range(n // C):
        ohk = oh[k * C:(k + 1) * C, :]
        pref = jnp.dot(L, ohk, preferred_element_type=jnp.float32) + carry
        chunks.append(pref)
        carry = carry + jnp.sum(ohk, axis=0, keepdims=True)
    prefix = jnp.concatenate(chunks, axis=0)
    counts = carry

    padded = jnp.ceil(counts / T_FFN) * T_FFN
    u_r = lax.broadcasted_iota(jnp.int32, (E, E), 0)
    u_c = lax.broadcasted_iota(jnp.int32, (E, E), 1)
    U = (u_r < u_c).astype(jnp.float32)
    starts = jnp.dot(padded, U, preferred_element_type=jnp.float32)
    ends = starts + padded

    rank0 = jnp.sum(prefix * oh0, axis=1, keepdims=True)
    rank1 = jnp.sum((prefix + oh0) * oh1, axis=1, keepdims=True)
    base0 = jnp.sum(starts * oh0, axis=1, keepdims=True)
    base1 = jnp.sum(starts * oh1, axis=1, keepdims=True)
    pos0 = (base0 + rank0).astype(jnp.int32)
    pos1 = (base1 + rank1).astype(jnp.int32)
    lane_out = lax.broadcasted_iota(jnp.int32, (n, E), 1)
    pos_ref[...] = jnp.where(lane_out == 0, pos0,
                             jnp.where(lane_out == 1, pos1, 0))

    n_tiles = (pos_ref.shape[0] * TOP_K + E * T_FFN) // T_FFN
    gl = lax.broadcasted_iota(jnp.int32, (1, 128), 1)
    ends_b = jnp.broadcast_to(ends.reshape(E, 1), (E, 128))
    te_row = jnp.minimum(
        jnp.sum((ends_b <= (gl * T_FFN).astype(jnp.float32)).astype(jnp.int32),
                axis=0, keepdims=True), E - 1)
    n_live = (ends[0, E - 1] / T_FFN).astype(jnp.int32)
    meta_ref[...] = jnp.where(gl == n_tiles, n_live, te_row)


def _ffn_body(te_ref, x_ref, w1_ref, b1_ref, w2_ref, b2_ref, y_ref):
    n_tiles = pl.num_programs(0)

    @pl.when(pl.program_id(0) < te_ref[n_tiles])
    def _():
        h = jnp.dot(x_ref[...], w1_ref[0], preferred_element_type=jnp.float32)
        h = h + b1_ref[0]
        h = 0.5 * h * (1.0 + jax.lax.erf(h * 0.7071067811865476))
        y = jnp.dot(h.astype(jnp.bfloat16), w2_ref[0],
                    preferred_element_type=jnp.float32)
        y_ref[...] = y + b2_ref[0]


def kernel(x, expert_mus, expert_log_sigmas, W1, b1, W2, b2):
    batch_size, num_tokens, _ = x.shape
    n = batch_size * num_tokens
    x_flat = x.reshape(n, D_IN)

    inv_sigmas = 1.0 / jnp.exp(expert_log_sigmas)
    lss_row = jnp.sum(expert_log_sigmas, axis=-1).reshape(1, E)

    n_rt = n // T_ROUTE
    lp, w, ti_pad, g_pad, xb = pl.pallas_call(
        _routing_body,
        grid=(n_rt,),
        in_specs=[
            pl.BlockSpec((T_ROUTE, D_IN), lambda i: (i, 0)),
            pl.BlockSpec((E, D_IN), lambda i: (0, 0)),
            pl.BlockSpec((E, D_IN), lambda i: (0, 0)),
            pl.BlockSpec((1, E), lambda i: (0, 0)),
        ],
        out_specs=[
            pl.BlockSpec((T_ROUTE, E), lambda i: (i, 0)),
            pl.BlockSpec((T_ROUTE, E), lambda i: (i, 0)),
            pl.BlockSpec((T_ROUTE, E), lambda i: (i, 0)),
            pl.BlockSpec((T_ROUTE, E), lambda i: (i, 0)),
            pl.BlockSpec((T_ROUTE, D_IN), lambda i: (i, 0)),
        ],
        out_shape=[
            jax.ShapeDtypeStruct((n, E), jnp.float32),
            jax.ShapeDtypeStruct((n, E), jnp.float32),
            jax.ShapeDtypeStruct((n, E), jnp.int32),
            jax.ShapeDtypeStruct((n, E), jnp.float32),
            jax.ShapeDtypeStruct((n, D_IN), jnp.bfloat16),
        ],
    )(x_flat, expert_mus, inv_sigmas, lss_row)

    top_indices = ti_pad[:, :TOP_K]
    gates = g_pad[:, :TOP_K]

    n_assign = n * TOP_K
    r_max = n_assign + E * T_FFN
    n_tiles = r_max // T_FFN

    pos_pad, meta_row = pl.pallas_call(
        _dispatch_body,
        out_shape=[
            jax.ShapeDtypeStruct((n, E), jnp.int32),
            jax.ShapeDtypeStruct((1, 128), jnp.int32),
        ],
    )(ti_pad)

    pos_a = pos_pad[:, :TOP_K].reshape(-1)
    token_a = jnp.arange(n_assign, dtype=jnp.int32) // TOP_K
    row_token = jnp.zeros((r_max,), jnp.int32).at[pos_a].set(token_a)
    tile_meta = meta_row[0, :n_tiles + 1]

    x_rows = xb[row_token]

    y = pl.pallas_call(
        _ffn_body,
        grid_spec=pltpu.PrefetchScalarGridSpec(
            num_scalar_prefetch=1,
            grid=(n_tiles,),
            in_specs=[
                pl.BlockSpec((T_FFN, D_IN), lambda g, te: (g, 0)),
                pl.BlockSpec((1, D_IN, D_H), lambda g, te: (te[g], 0, 0)),
                pl.BlockSpec((1, 1, D_H), lambda g, te: (te[g], 0, 0)),
                pl.BlockSpec((1, D_H, D_OUT), lambda g, te: (te[g], 0, 0)),
                pl.BlockSpec((1, 1, D_OUT), lambda g, te: (te[g], 0, 0)),
            ],
            out_specs=pl.BlockSpec((T_FFN, D_OUT), lambda g, te: (g, 0)),
        ),
        out_shape=jax.ShapeDtypeStruct((r_max, D_OUT), jnp.float32),
    )(tile_meta, x_rows, W1.astype(jnp.bfloat16), b1.reshape(E, 1, D_H),
      W2.astype(jnp.bfloat16), b2.reshape(E, 1, D_OUT))

    p0 = pos_a[0::TOP_K]
    p1 = pos_a[1::TOP_K]
    final = gates[:, 0:1] * y[p0] + gates[:, 1:2] * y[p1]

    return (final.reshape(batch_size, num_tokens, D_OUT),
            lp.reshape(batch_size, num_tokens, E),
            w.reshape(batch_size, num_tokens, E),
            top_indices)


# speedup vs baseline: 1.0291x; 1.0291x over previous
import jax
import jax.numpy as jnp
from jax import lax
from jax.experimental import pallas as pl
from jax.experimental.pallas import tpu as pltpu

E = 8
TOP_K = 2
D_IN = 1024
D_H = 4096
D_OUT = 1024

T_FFN = 256
T_ROUTE = 512


def _routing_body(x_ref, mus_ref, sig_ref, lss_ref, lp_ref, w_ref, ti_ref, g_ref,
                  xb_ref):
    x = x_ref[...]
    rows = x.shape[0]
    d = jnp.zeros((rows, E), jnp.float32)
    eidx = jax.lax.broadcasted_iota(jnp.int32, (rows, E), 1)
    for e in range(E):
        t = (x - mus_ref[pl.ds(e, 1), :]) * sig_ref[pl.ds(e, 1), :]
        de = jnp.sum(t * t, axis=1, keepdims=True)
        d = jnp.where(eidx == e, de, d)
    lp = -0.5 * d - lss_ref[...]
    m = jnp.max(lp, axis=1, keepdims=True)
    ew = jnp.exp(lp - m)
    w = ew / jnp.sum(ew, axis=1, keepdims=True)

    iota = jax.lax.broadcasted_iota(jnp.int32, w.shape, 1)
    m1 = jnp.max(w, axis=1, keepdims=True)
    i1 = jnp.min(jnp.where(w == m1, iota, E), axis=1, keepdims=True)
    wm = jnp.where(iota == i1, -jnp.inf, w)
    m2 = jnp.max(wm, axis=1, keepdims=True)
    i2 = jnp.min(jnp.where(wm == m2, iota, E), axis=1, keepdims=True)

    lp_ref[...] = lp
    w_ref[...] = w
    ti_ref[...] = jnp.where(iota == 0, i1, jnp.where(iota == 1, i2, 0))
    g_ref[...] = jnp.where(iota == 0, m1, jnp.where(iota == 1, m2, 0.0))
    xb_ref[...] = x.astype(jnp.bfloat16)


def _dispatch_body(ti_ref, pos_ref, meta_ref):
    C = 512
    n = ti_ref.shape[0]
    e0 = ti_ref[:, 0:1]
    e1 = ti_ref[:, 1:2]
    lane8 = lax.broadcasted_iota(jnp.int32, (n, E), 1)
    oh0 = (e0 == lane8).astype(jnp.float32)
    oh1 = (e1 == lane8).astype(jnp.float32)
    oh = oh0 + oh1

    r = lax.broadcasted_iota(jnp.int32, (C, C), 0)
    c = lax.broadcasted_iota(jnp.int32, (C, C), 1)
    L = (r > c).astype(jnp.float32)

    carry = jnp.zeros((1, E), jnp.float32)
    chunks = []
    for k in range(n // C):
        ohk = oh[k * C:(k + 1) * C, :]
        pref = jnp.dot(L, ohk, preferred_element_type=jnp.float32) + carry
        chunks.append(pref)
        carry = carry + jnp.sum(ohk, axis=0, keepdims=True)
    prefix = jnp.concatenate(chunks, axis=0)
    counts = carry

    padded = jnp.ceil(counts / T_FFN) * T_FFN
    u_r = lax.broadcasted_iota(jnp.int32, (E, E), 0)
    u_c = lax.broadcasted_iota(jnp.int32, (E, E), 1)
    U = (u_r < u_c).astype(jnp.float32)
    starts = jnp.dot(padded, U, preferred_element_type=jnp.float32)
    ends = starts + padded

    rank0 = jnp.sum(prefix * oh0, axis=1, keepdims=True)
    rank1 = jnp.sum((prefix + oh0) * oh1, axis=1, keepdims=True)
    base0 = jnp.sum(starts * oh0, axis=1, keepdims=True)
    base1 = jnp.sum(starts * oh1, axis=1, keepdims=True)
    pos0 = (base0 + rank0).astype(jnp.int32)
    pos1 = (base1 + rank1).astype(jnp.int32)
    lane_out = lax.broadcasted_iota(jnp.int32, (n, E), 1)
    pos_ref[...] = jnp.where(lane_out == 0, pos0,
                             jnp.where(lane_out == 1, pos1, 0))

    n_tiles = (pos_ref.shape[0] * TOP_K + E * T_FFN) // T_FFN
    gl = lax.broadcasted_iota(jnp.int32, (1, 128), 1)
    ends_b = jnp.broadcast_to(ends.reshape(E, 1), (E, 128))
    te_row = jnp.minimum(
        jnp.sum((ends_b <= (gl * T_FFN).astype(jnp.float32)).astype(jnp.int32),
                axis=0, keepdims=True), E - 1)
    n_live = (ends[0, E - 1] / T_FFN).astype(jnp.int32)
    meta_ref[...] = jnp.where(gl == n_tiles, n_live, te_row)


def _ffn_body(te_ref, x_ref, w1_ref, b1_ref, w2_ref, b2_ref, y_ref):
    n_tiles = pl.num_programs(0)

    @pl.when(pl.program_id(0) < te_ref[n_tiles])
    def _():
        h = jnp.dot(x_ref[...], w1_ref[0], preferred_element_type=jnp.float32)
        h = h + b1_ref[0]
        h = 0.5 * h * (1.0 + jax.lax.erf(h * 0.7071067811865476))
        y = jnp.dot(h.astype(jnp.bfloat16), w2_ref[0],
                    preferred_element_type=jnp.float32)
        y_ref[...] = y + b2_ref[0]


def kernel(x, expert_mus, expert_log_sigmas, W1, b1, W2, b2):
    batch_size, num_tokens, _ = x.shape
    n = batch_size * num_tokens
    x_flat = x.reshape(n, D_IN)

    inv_sigmas = 1.0 / jnp.exp(expert_log_sigmas)
    lss_row = jnp.sum(expert_log_sigmas, axis=-1).reshape(1, E)

    n_rt = n // T_ROUTE
    lp, w, ti_pad, g_pad, xb = pl.pallas_call(
        _routing_body,
        grid=(n_rt,),
        in_specs=[
            pl.BlockSpec((T_ROUTE, D_IN), lambda i: (i, 0)),
            pl.BlockSpec((E, D_IN), lambda i: (0, 0)),
            pl.BlockSpec((E, D_IN), lambda i: (0, 0)),
            pl.BlockSpec((1, E), lambda i: (0, 0)),
        ],
        out_specs=[
            pl.BlockSpec((T_ROUTE, E), lambda i: (i, 0)),
            pl.BlockSpec((T_ROUTE, E), lambda i: (i, 0)),
            pl.BlockSpec((T_ROUTE, E), lambda i: (i, 0)),
            pl.BlockSpec((T_ROUTE, E), lambda i: (i, 0)),
            pl.BlockSpec((T_ROUTE, D_IN), lambda i: (i, 0)),
        ],
        out_shape=[
            jax.ShapeDtypeStruct((n, E), jnp.float32),
            jax.ShapeDtypeStruct((n, E), jnp.float32),
            jax.ShapeDtypeStruct((n, E), jnp.int32),
            jax.ShapeDtypeStruct((n, E), jnp.float32),
            jax.ShapeDtypeStruct((n, D_IN), jnp.bfloat16),
        ],
    )(x_flat, expert_mus, inv_sigmas, lss_row)

    top_indices = ti_pad[:, :TOP_K]
    gates = g_pad[:, :TOP_K]

    n_assign = n * TOP_K
    r_max = n_assign + E * T_FFN
    n_tiles = r_max // T_FFN

    pos_pad, meta_row = pl.pallas_call(
        _dispatch_body,
        out_shape=[
            jax.ShapeDtypeStruct((n, E), jnp.int32),
            jax.ShapeDtypeStruct((1, 128), jnp.int32),
        ],
    )(ti_pad)

    pos_a = pos_pad[:, :TOP_K].reshape(-1)
    token_a = jnp.arange(n_assign, dtype=jnp.int32) // TOP_K
    row_token = jnp.zeros((r_max,), jnp.int32).at[pos_a].set(token_a)
    tile_meta = meta_row[0, :n_tiles + 1]

    x_rows = xb[row_token]

    y = pl.pallas_call(
        _ffn_body,
        grid_spec=pltpu.PrefetchScalarGridSpec(
            num_scalar_prefetch=1,
            grid=(n_tiles,),
            in_specs=[
                pl.BlockSpec((T_FFN, D_IN), lambda g, te: (g, 0)),
                pl.BlockSpec((1, D_IN, D_H), lambda g, te: (te[g], 0, 0)),
                pl.BlockSpec((1, 1, D_H), lambda g, te: (te[g], 0, 0)),
                pl.BlockSpec((1, D_H, D_OUT), lambda g, te: (te[g], 0, 0)),
                pl.BlockSpec((1, 1, D_OUT), lambda g, te: (te[g], 0, 0)),
            ],
            out_specs=pl.BlockSpec((T_FFN, D_OUT), lambda g, te: (g, 0)),
        ),
        out_shape=jax.ShapeDtypeStruct((r_max, D_OUT), jnp.float32),
    )(tile_meta, x_rows, W1.astype(jnp.bfloat16), b1.reshape(E, 1, D_H),
      W2.astype(jnp.bfloat16), b2.reshape(E, 1, D_OUT))

    p0 = pos_a[0::TOP_K]
    p1 = pos_a[1::TOP_K]
    final = gates[:, 0:1] * y[p0] + gates[:, 1:2] * y[p1]

    return (final.reshape(batch_size, num_tokens, D_OUT),
            lp.reshape(batch_size, num_tokens, E),
            w.reshape(batch_size, num_tokens, E),
            top_indices)
